# Initial kernel scaffold; baseline (speedup 1.0000x reference)
import jax
import jax.numpy as jnp
from jax.experimental import pallas as pl


def kernel(x, edge_index, W, b):
    raise NotImplementedError("write your pallas kernel here")


# speedup vs baseline: 16.3060x; 16.3060x over previous
import functools

import jax
import jax.numpy as jnp
from jax import lax
from jax.experimental import pallas as pl
from jax.experimental.pallas import tpu as pltpu
from jax.experimental.pallas import tpu_sc as plsc

N = 10000
E = 320000
D = 128
NC = 2
NS = 16
NW = NC * NS
EPW = E // NW
CH = 80
NIT = EPW // CH
KB = 5
ZCH = 200
NCHK = N // ZCH
KMAX = -(-NCHK // NS)
DW = 128


def _sc_degree(col, ones, zeros):
    mesh = plsc.VectorSubcoreMesh(core_axis_name="c", subcore_axis_name="s", num_cores=NC, num_subcores=NS)

    @functools.partial(
        pl.kernel,
        out_type=jax.ShapeDtypeStruct((NC, N, DW), jnp.float32),
        mesh=mesh,
        scratch_types=[
            pltpu.VMEM((KB, CH), jnp.int32),
            pltpu.VMEM((CH, DW), jnp.float32),
            pltpu.VMEM((ZCH, DW), jnp.float32),
            pltpu.VMEM_SHARED((N, DW), jnp.float32),
            pltpu.SemaphoreType.DMA,
        ],
    )
    def deg_kernel(col_hbm, ones_hbm, zeros_hbm, out_hbm,
                   idx_v, ones_v, zbuf_v, deg_sh, fsem):
        c = lax.axis_index("c")
        s = lax.axis_index("s")
        wid = s * NC + c

        pltpu.sync_copy(ones_hbm, ones_v)
        pltpu.sync_copy(zeros_hbm, zbuf_v)

        def zchunk(k, _):
            ci = s + NS * k

            @pl.when(ci < NCHK)
            def _():
                pltpu.sync_copy(zbuf_v, deg_sh.at[pl.ds(ZCH * ci, ZCH)])

            return 0

        lax.fori_loop(0, KMAX, zchunk, 0)
        plsc.subcore_barrier()

        def edge(i, _):
            base = wid * EPW + i * (KB * CH)
            for j in range(KB):
                pltpu.sync_copy(col_hbm.at[pl.ds(base + j * CH, CH)],
                                idx_v.at[j])
            cps = [pltpu.make_async_copy(ones_v, deg_sh.at[idx_v.at[j]], fsem)
                   for j in range(KB)]
            for cp in cps:
                cp.start(add=True)
            for cp in cps:
                cp.wait()
            return 0

        lax.fori_loop(0, NIT // KB, edge, 0)
        plsc.subcore_barrier()

        def cout(k, _):
            ci = s + NS * k

            @pl.when(ci < NCHK)
            def _():
                r = ZCH * ci
                pltpu.sync_copy(deg_sh.at[pl.ds(r, ZCH)],
                                out_hbm.at[c, pl.ds(r, ZCH)])

            return 0

        lax.fori_loop(0, KMAX, cout, 0)

    return deg_kernel(col, ones, zeros)


def _sc_scatter(y, row, col, zeros):
    mesh = plsc.VectorSubcoreMesh(core_axis_name="c", subcore_axis_name="s", num_cores=NC, num_subcores=NS)

    @functools.partial(
        pl.kernel,
        out_type=jax.ShapeDtypeStruct((NC, N, D), jnp.float32),
        mesh=mesh,
        scratch_types=[
            pltpu.VMEM((CH,), jnp.int32),
            pltpu.VMEM((CH,), jnp.int32),
            pltpu.VMEM((CH, D), jnp.float32),
            pltpu.VMEM((ZCH, D), jnp.float32),
            pltpu.VMEM_SHARED((N, D), jnp.float32),
            pltpu.SemaphoreType.DMA,
        ],
    )
    def scat_kernel(y_hbm, row_hbm, col_hbm, zeros_hbm, out_hbm,
                    ridx_v, cidx_v, rows_v, zbuf_v, acc_sh, sem):
        c = lax.axis_index("c")
        s = lax.axis_index("s")
        wid = s * NC + c

        pltpu.sync_copy(zeros_hbm, zbuf_v)

        def zchunk(k, _):
            ci = s + NS * k

            @pl.when(ci < NCHK)
            def _():
                pltpu.sync_copy(zbuf_v, acc_sh.at[pl.ds(ZCH * ci, ZCH)])

            return 0

        lax.fori_loop(0, KMAX, zchunk, 0)
        plsc.subcore_barrier()

        def edge(i, _):
            base = wid * EPW + i * CH
            pltpu.sync_copy(row_hbm.at[pl.ds(base, CH)], ridx_v)
            pltpu.sync_copy(col_hbm.at[pl.ds(base, CH)], cidx_v)
            pltpu.async_copy(y_hbm.at[ridx_v], rows_v, sem).wait()
            pltpu.sync_copy(rows_v, acc_sh.at[cidx_v], add=True)
            return 0

        lax.fori_loop(0, NIT, edge, 0)
        plsc.subcore_barrier()

        def cout(k, _):
            ci = s + NS * k

            @pl.when(ci < NCHK)
            def _():
                r = ZCH * ci
                pltpu.sync_copy(acc_sh.at[pl.ds(r, ZCH)],
                                out_hbm.at[c, pl.ds(r, ZCH)])

            return 0

        lax.fori_loop(0, KMAX, cout, 0)

    return scat_kernel(y, row, col, zeros)


_GRID = 10
_B = N // _GRID


def _dis_block(deg_ref):
    d = jnp.sum(deg_ref[0] + deg_ref[1], axis=1, keepdims=True) + 1.0
    return lax.rsqrt(d)


def _tc_matmul(x, W, deg2):
    def body(x_ref, w_ref, deg_ref, y_ref):
        dis = _dis_block(deg_ref)
        xw = jnp.dot(x_ref[...], w_ref[...],
                     preferred_element_type=jnp.float32)
        y_ref[...] = xw * dis

    return pl.pallas_call(
        body,
        grid=(_GRID,),
        in_specs=[
            pl.BlockSpec((_B, D), lambda i: (i, 0)),
            pl.BlockSpec((D, D), lambda i: (0, 0)),
            pl.BlockSpec((NC, _B, DW), lambda i: (0, i, 0)),
        ],
        out_specs=pl.BlockSpec((_B, D), lambda i: (i, 0)),
        out_shape=jax.ShapeDtypeStruct((N, D), jnp.float32),
    )(x, W, deg2)


def _tc_finish(acc, y, deg2, b2):
    def body(acc_ref, y_ref, deg_ref, b_ref, o_ref):
        dis = _dis_block(deg_ref)
        su = (acc_ref[0] + acc_ref[1] + y_ref[...]) * dis + b_ref[...]
        o_ref[...] = jnp.maximum(su, 0.0)

    return pl.pallas_call(
        body,
        grid=(_GRID,),
        in_specs=[
            pl.BlockSpec((NC, _B, D), lambda i: (0, i, 0)),
            pl.BlockSpec((_B, D), lambda i: (i, 0)),
            pl.BlockSpec((NC, _B, DW), lambda i: (0, i, 0)),
            pl.BlockSpec((1, D), lambda i: (0, 0)),
        ],
        out_specs=pl.BlockSpec((_B, D), lambda i: (i, 0)),
        out_shape=jax.ShapeDtypeStruct((N, D), jnp.float32),
    )(acc, y, deg2, b2)


def kernel(x, edge_index, W, b):
    row = edge_index[0].astype(jnp.int32)
    col = edge_index[1].astype(jnp.int32)
    ones = jnp.full((CH, DW), 1.0 / DW, jnp.float32)
    zeros = jnp.zeros((ZCH, D), jnp.float32)
    deg2 = _sc_degree(col, ones, zeros)
    y = _tc_matmul(x, W, deg2)
    acc = _sc_scatter(y, row, col, zeros)
    return _tc_finish(acc, y, deg2, jnp.reshape(b, (1, D)))


# speedup vs baseline: 23.4234x; 1.4365x over previous
import functools

import jax
import jax.numpy as jnp
from jax import lax
from jax.experimental import pallas as pl
from jax.experimental.pallas import tpu as pltpu
from jax.experimental.pallas import tpu_sc as plsc

N = 10000
E = 320000
D = 128
NC = 2
NS = 16
NW = NC * NS
EPW = E // NW
CH = 80
NIT = EPW // CH
KB = 5
ZCS = 80
ZCH = 200
NCHK = N // ZCH
KMAX = -(-NCHK // NS)
DW = 128


def _sc_degree(col, ones, zeros):
    mesh = plsc.VectorSubcoreMesh(core_axis_name="c", subcore_axis_name="s", num_cores=NC, num_subcores=NS)

    @functools.partial(
        pl.kernel,
        out_type=jax.ShapeDtypeStruct((NC, N, DW), jnp.float32),
        mesh=mesh,
        scratch_types=[
            pltpu.VMEM((KB, CH), jnp.int32),
            pltpu.VMEM((CH, DW), jnp.float32),
            pltpu.VMEM((ZCH, DW), jnp.float32),
            pltpu.VMEM_SHARED((N, DW), jnp.float32),
            pltpu.SemaphoreType.DMA,
        ],
    )
    def deg_kernel(col_hbm, ones_hbm, zeros_hbm, out_hbm,
                   idx_v, ones_v, zbuf_v, deg_sh, fsem):
        c = lax.axis_index("c")
        s = lax.axis_index("s")
        wid = s * NC + c

        pltpu.sync_copy(ones_hbm, ones_v)
        pltpu.sync_copy(zeros_hbm, zbuf_v)

        def zchunk(k, _):
            ci = s + NS * k

            @pl.when(ci < NCHK)
            def _():
                pltpu.sync_copy(zbuf_v, deg_sh.at[pl.ds(ZCH * ci, ZCH)])

            return 0

        lax.fori_loop(0, KMAX, zchunk, 0)
        plsc.subcore_barrier()

        def edge(i, _):
            base = wid * EPW + i * (KB * CH)
            for j in range(KB):
                pltpu.sync_copy(col_hbm.at[pl.ds(base + j * CH, CH)],
                                idx_v.at[j])
            cps = [pltpu.make_async_copy(ones_v, deg_sh.at[idx_v.at[j]], fsem)
                   for j in range(KB)]
            for cp in cps:
                cp.start(add=True)
            for cp in cps:
                cp.wait()
            return 0

        lax.fori_loop(0, NIT // KB, edge, 0)
        plsc.subcore_barrier()

        def cout(k, _):
            ci = s + NS * k

            @pl.when(ci < NCHK)
            def _():
                r = ZCH * ci
                pltpu.sync_copy(deg_sh.at[pl.ds(r, ZCH)],
                                out_hbm.at[c, pl.ds(r, ZCH)])

            return 0

        lax.fori_loop(0, KMAX, cout, 0)

    return deg_kernel(col, ones, zeros)


def _sc_scatter(y, row, col, zeros):
    mesh = plsc.VectorSubcoreMesh(core_axis_name="c", subcore_axis_name="s", num_cores=NC, num_subcores=NS)

    @functools.partial(
        pl.kernel,
        out_type=jax.ShapeDtypeStruct((NC, N, D), jnp.float32),
        mesh=mesh,
        scratch_types=[
            pltpu.VMEM((KB, CH), jnp.int32),
            pltpu.VMEM((KB, CH), jnp.int32),
            pltpu.VMEM((2, CH, D), jnp.float32),
            pltpu.VMEM((ZCS, D), jnp.float32),
            pltpu.VMEM_SHARED((N, D), jnp.float32),
            pltpu.SemaphoreType.DMA,
            pltpu.SemaphoreType.DMA,
        ],
    )
    def scat_kernel(y_hbm, row_hbm, col_hbm, zeros_hbm, out_hbm,
                    ridx_v, cidx_v, rows_v, zbuf_v, acc_sh, gsem, ssem):
        c = lax.axis_index("c")
        s = lax.axis_index("s")
        wid = s * NC + c

        pltpu.sync_copy(zeros_hbm, zbuf_v)

        def zchunk(k, _):
            ci = s + NS * k

            @pl.when(ci < N // ZCS)
            def _():
                pltpu.sync_copy(zbuf_v, acc_sh.at[pl.ds(ZCS * ci, ZCS)])

            return 0

        lax.fori_loop(0, -(-(N // ZCS) // NS), zchunk, 0)
        plsc.subcore_barrier()

        def edge(g, _):
            pltpu.sync_copy(row_hbm.at[wid, g], ridx_v)
            pltpu.sync_copy(col_hbm.at[wid, g], cidx_v)
            gcps = [pltpu.make_async_copy(y_hbm.at[ridx_v.at[j]],
                                          rows_v.at[j % 2], gsem)
                    for j in range(KB)]
            scps = [pltpu.make_async_copy(rows_v.at[j % 2],
                                          acc_sh.at[cidx_v.at[j]], ssem)
                    for j in range(KB)]
            gcps[0].start()
            gcps[1].start()
            for j in range(KB):
                gcps[j].wait()
                scps[j].start(add=True)
                scps[j].wait()
                if j + 2 < KB:
                    gcps[j + 2].start()
            return 0

        lax.fori_loop(0, NIT // KB, edge, 0)
        plsc.subcore_barrier()

        def cout(k, _):
            ci = s + NS * k

            @pl.when(ci < NCHK)
            def _():
                r = ZCH * ci
                pltpu.sync_copy(acc_sh.at[pl.ds(r, ZCH)],
                                out_hbm.at[c, pl.ds(r, ZCH)])

            return 0

        lax.fori_loop(0, KMAX, cout, 0)

    row4 = jnp.reshape(row, (NW, NIT // KB, KB, CH))
    col4 = jnp.reshape(col, (NW, NIT // KB, KB, CH))
    return scat_kernel(y, row4, col4, zeros[:ZCS])


_GRID = 10
_B = N // _GRID


def _dis_block(deg_ref):
    d = jnp.sum(deg_ref[0] + deg_ref[1], axis=1, keepdims=True) + 1.0
    return lax.rsqrt(d)


def _tc_matmul(x, W, deg2):
    def body(x_ref, w_ref, deg_ref, y_ref):
        dis = _dis_block(deg_ref)
        xw = jnp.dot(x_ref[...], w_ref[...],
                     preferred_element_type=jnp.float32)
        y_ref[...] = xw * dis

    return pl.pallas_call(
        body,
        grid=(_GRID,),
        in_specs=[
            pl.BlockSpec((_B, D), lambda i: (i, 0)),
            pl.BlockSpec((D, D), lambda i: (0, 0)),
            pl.BlockSpec((NC, _B, DW), lambda i: (0, i, 0)),
        ],
        out_specs=pl.BlockSpec((_B, D), lambda i: (i, 0)),
        out_shape=jax.ShapeDtypeStruct((N, D), jnp.float32),
    )(x, W, deg2)


def _tc_finish(acc, y, deg2, b2):
    def body(acc_ref, y_ref, deg_ref, b_ref, o_ref):
        dis = _dis_block(deg_ref)
        su = (acc_ref[0] + acc_ref[1] + y_ref[...]) * dis + b_ref[...]
        o_ref[...] = jnp.maximum(su, 0.0)

    return pl.pallas_call(
        body,
        grid=(_GRID,),
        in_specs=[
            pl.BlockSpec((NC, _B, D), lambda i: (0, i, 0)),
            pl.BlockSpec((_B, D), lambda i: (i, 0)),
            pl.BlockSpec((NC, _B, DW), lambda i: (0, i, 0)),
            pl.BlockSpec((1, D), lambda i: (0, 0)),
        ],
        out_specs=pl.BlockSpec((_B, D), lambda i: (i, 0)),
        out_shape=jax.ShapeDtypeStruct((N, D), jnp.float32),
    )(acc, y, deg2, b2)


def kernel(x, edge_index, W, b):
    row = edge_index[0].astype(jnp.int32)
    col = edge_index[1].astype(jnp.int32)
    ones = jnp.full((CH, DW), 1.0 / DW, jnp.float32)
    zeros = jnp.zeros((ZCH, D), jnp.float32)
    deg2 = _sc_degree(col, ones, zeros)
    y = _tc_matmul(x, W, deg2)
    acc = _sc_scatter(y, row, col, zeros)
    return _tc_finish(acc, y, deg2, jnp.reshape(b, (1, D)))


# speedup vs baseline: 23.4894x; 1.0028x over previous
import functools

import jax
import jax.numpy as jnp
from jax import lax
from jax.experimental import pallas as pl
from jax.experimental.pallas import tpu as pltpu
from jax.experimental.pallas import tpu_sc as plsc

N = 10000
E = 320000
D = 128
NC = 2
NS = 16
NW = NC * NS
EPW = E // NW
CH = 80
NIT = EPW // CH
KB = 5
ZCS = 80
ZCH = 200
NCHK = N // ZCH
KMAX = -(-NCHK // NS)
DW = 128


def _sc_degree(col, ones, zeros):
    mesh = plsc.VectorSubcoreMesh(core_axis_name="c", subcore_axis_name="s", num_cores=NC, num_subcores=NS)

    @functools.partial(
        pl.kernel,
        out_type=jax.ShapeDtypeStruct((NC, N, DW), jnp.float32),
        mesh=mesh,
        scratch_types=[
            pltpu.VMEM((KB, CH), jnp.int32),
            pltpu.VMEM((CH, DW), jnp.float32),
            pltpu.VMEM((ZCH, DW), jnp.float32),
            pltpu.VMEM_SHARED((N, DW), jnp.float32),
            pltpu.SemaphoreType.DMA,
        ],
    )
    def deg_kernel(col_hbm, ones_hbm, zeros_hbm, out_hbm,
                   idx_v, ones_v, zbuf_v, deg_sh, fsem):
        c = lax.axis_index("c")
        s = lax.axis_index("s")
        wid = s * NC + c

        pltpu.sync_copy(ones_hbm, ones_v)
        pltpu.sync_copy(zeros_hbm, zbuf_v)

        def zchunk(k, _):
            ci = s + NS * k

            @pl.when(ci < NCHK)
            def _():
                pltpu.sync_copy(zbuf_v, deg_sh.at[pl.ds(ZCH * ci, ZCH)])

            return 0

        lax.fori_loop(0, KMAX, zchunk, 0)
        plsc.subcore_barrier()

        def edge(i, _):
            base = wid * EPW + i * (KB * CH)
            for j in range(KB):
                pltpu.sync_copy(col_hbm.at[pl.ds(base + j * CH, CH)],
                                idx_v.at[j])
            cps = [pltpu.make_async_copy(ones_v, deg_sh.at[idx_v.at[j]], fsem)
                   for j in range(KB)]
            for cp in cps:
                cp.start(add=True)
            for cp in cps:
                cp.wait()
            return 0

        lax.fori_loop(0, NIT // KB, edge, 0)
        plsc.subcore_barrier()

        def cout(k, _):
            ci = s + NS * k

            @pl.when(ci < NCHK)
            def _():
                r = ZCH * ci
                pltpu.sync_copy(deg_sh.at[pl.ds(r, ZCH)],
                                out_hbm.at[c, pl.ds(r, ZCH)])

            return 0

        lax.fori_loop(0, KMAX, cout, 0)

    return deg_kernel(col, ones, zeros)


def _sc_scatter(y, row, col, zeros):
    mesh = plsc.VectorSubcoreMesh(core_axis_name="c", subcore_axis_name="s", num_cores=NC, num_subcores=NS)

    @functools.partial(
        pl.kernel,
        out_type=jax.ShapeDtypeStruct((NC, N, D), jnp.float32),
        mesh=mesh,
        scratch_types=[
            pltpu.VMEM((KB, CH), jnp.int32),
            pltpu.VMEM((KB, CH), jnp.int32),
            pltpu.VMEM((2, CH, D), jnp.float32),
            pltpu.VMEM((ZCS, D), jnp.float32),
            pltpu.VMEM_SHARED((N, D), jnp.float32),
            pltpu.SemaphoreType.DMA,
            pltpu.SemaphoreType.DMA,
        ],
    )
    def scat_kernel(y_hbm, row_hbm, col_hbm, zeros_hbm, out_hbm,
                    ridx_v, cidx_v, rows_v, zbuf_v, acc_sh, gsem, ssem):
        c = lax.axis_index("c")
        s = lax.axis_index("s")
        wid = s * NC + c

        pltpu.sync_copy(zeros_hbm, zbuf_v)

        def zchunk(k, _):
            ci = s + NS * k

            @pl.when(ci < N // ZCS)
            def _():
                pltpu.sync_copy(zbuf_v, acc_sh.at[pl.ds(ZCS * ci, ZCS)])

            return 0

        lax.fori_loop(0, -(-(N // ZCS) // NS), zchunk, 0)
        plsc.subcore_barrier()

        def edge(g, _):
            pltpu.sync_copy(row_hbm.at[wid, g], ridx_v)
            pltpu.sync_copy(col_hbm.at[wid, g], cidx_v)
            gcps = [pltpu.make_async_copy(y_hbm.at[ridx_v.at[j]],
                                          rows_v.at[j % 2], gsem)
                    for j in range(KB)]
            scps = [pltpu.make_async_copy(rows_v.at[j % 2],
                                          acc_sh.at[cidx_v.at[j]], ssem)
                    for j in range(KB)]
            gcps[0].start()
            gcps[1].start()
            for j in range(KB):
                gcps[j].wait()
                scps[j].start(add=True)
                scps[j].wait()
                if j + 2 < KB:
                    gcps[j + 2].start()
            return 0

        lax.fori_loop(0, NIT // KB, edge, 0)
        plsc.subcore_barrier()

        def cout(k, _):
            ci = s + NS * k

            @pl.when(ci < NCHK)
            def _():
                r = ZCH * ci
                pltpu.sync_copy(acc_sh.at[pl.ds(r, ZCH)],
                                out_hbm.at[c, pl.ds(r, ZCH)])

            return 0

        lax.fori_loop(0, KMAX, cout, 0)

    row4 = jnp.reshape(row, (NW, NIT // KB, KB, CH))
    col4 = jnp.reshape(col, (NW, NIT // KB, KB, CH))
    return scat_kernel(y, row4, col4, zeros[:ZCS])


_GRID = 10
_B = N // _GRID


def _dis_block(deg_ref):
    d = jnp.sum(deg_ref[0] + deg_ref[1], axis=1, keepdims=True) + 1.0
    return lax.rsqrt(d)


def _tc_xw(x, W):
    def body(x_ref, w_ref, y_ref):
        y_ref[...] = jnp.dot(x_ref[...], w_ref[...],
                             preferred_element_type=jnp.float32)

    return pl.pallas_call(
        body,
        grid=(_GRID,),
        in_specs=[
            pl.BlockSpec((_B, D), lambda i: (i, 0)),
            pl.BlockSpec((D, D), lambda i: (0, 0)),
        ],
        out_specs=pl.BlockSpec((_B, D), lambda i: (i, 0)),
        out_shape=jax.ShapeDtypeStruct((N, D), jnp.float32),
    )(x, W)


def _tc_scale(xw, deg2):
    def body(xw_ref, deg_ref, y_ref):
        y_ref[...] = xw_ref[...] * _dis_block(deg_ref)

    return pl.pallas_call(
        body,
        grid=(_GRID,),
        in_specs=[
            pl.BlockSpec((_B, D), lambda i: (i, 0)),
            pl.BlockSpec((NC, _B, DW), lambda i: (0, i, 0)),
        ],
        out_specs=pl.BlockSpec((_B, D), lambda i: (i, 0)),
        out_shape=jax.ShapeDtypeStruct((N, D), jnp.float32),
    )(xw, deg2)


def _tc_finish(acc, y, deg2, b2):
    def body(acc_ref, y_ref, deg_ref, b_ref, o_ref):
        dis = _dis_block(deg_ref)
        su = (acc_ref[0] + acc_ref[1] + y_ref[...]) * dis + b_ref[...]
        o_ref[...] = jnp.maximum(su, 0.0)

    return pl.pallas_call(
        body,
        grid=(_GRID,),
        in_specs=[
            pl.BlockSpec((NC, _B, D), lambda i: (0, i, 0)),
            pl.BlockSpec((_B, D), lambda i: (i, 0)),
            pl.BlockSpec((NC, _B, DW), lambda i: (0, i, 0)),
            pl.BlockSpec((1, D), lambda i: (0, 0)),
        ],
        out_specs=pl.BlockSpec((_B, D), lambda i: (i, 0)),
        out_shape=jax.ShapeDtypeStruct((N, D), jnp.float32),
    )(acc, y, deg2, b2)


def kernel(x, edge_index, W, b):
    row = edge_index[0].astype(jnp.int32)
    col = edge_index[1].astype(jnp.int32)
    ones = jnp.full((CH, DW), 1.0 / DW, jnp.float32)
    zeros = jnp.zeros((ZCH, D), jnp.float32)
    deg2 = _sc_degree(col, ones, zeros)
    xw = _tc_xw(x, W)
    y = _tc_scale(xw, deg2)
    acc = _sc_scatter(y, row, col, zeros)
    return _tc_finish(acc, y, deg2, jnp.reshape(b, (1, D)))


# speedup vs baseline: 26.3032x; 1.1198x over previous
import functools

import jax
import jax.numpy as jnp
from jax import lax
from jax.experimental import pallas as pl
from jax.experimental.pallas import tpu as pltpu
from jax.experimental.pallas import tpu_sc as plsc

N = 10000
E = 320000
D = 128
NC = 2
NS = 16
NW = NC * NS
EPW = E // NW
CH = 80
NIT = EPW // CH
KB = 5
ZCS = 80
ZCH = 200
NCHK = N // ZCH
KMAX = -(-NCHK // NS)
DW = 128
DWO = 16


def _sc_degree(col, ones, zeros):
    mesh = plsc.VectorSubcoreMesh(core_axis_name="c", subcore_axis_name="s", num_cores=NC, num_subcores=NS)

    @functools.partial(
        pl.kernel,
        out_type=jax.ShapeDtypeStruct((NC, N, DW), jnp.float32),
        mesh=mesh,
        scratch_types=[
            pltpu.VMEM((KB, CH), jnp.int32),
            pltpu.VMEM((CH, DW), jnp.float32),
            pltpu.VMEM((ZCH, DW), jnp.float32),
            pltpu.VMEM_SHARED((N, DW), jnp.float32),
            pltpu.SemaphoreType.DMA,
        ],
    )
    def deg_kernel(col_hbm, ones_hbm, zeros_hbm, out_hbm,
                   idx_v, ones_v, zbuf_v, deg_sh, fsem):
        c = lax.axis_index("c")
        s = lax.axis_index("s")
        wid = s * NC + c

        pltpu.sync_copy(ones_hbm, ones_v)
        pltpu.sync_copy(zeros_hbm, zbuf_v)

        def zchunk(k, _):
            ci = s + NS * k

            @pl.when(ci < NCHK)
            def _():
                pltpu.sync_copy(zbuf_v, deg_sh.at[pl.ds(ZCH * ci, ZCH)])

            return 0

        lax.fori_loop(0, KMAX, zchunk, 0)
        plsc.subcore_barrier()

        def edge(g, _):
            pltpu.sync_copy(col_hbm.at[wid, g], idx_v)
            cps = [pltpu.make_async_copy(ones_v, deg_sh.at[idx_v.at[j]], fsem)
                   for j in range(KB)]
            for cp in cps:
                cp.start(add=True)
            for cp in cps:
                cp.wait()
            return 0

        lax.fori_loop(0, NIT // KB, edge, 0)
        plsc.subcore_barrier()

        def cout(k, _):
            ci = s + NS * k

            @pl.when(ci < NCHK)
            def _():
                r = ZCH * ci
                pltpu.sync_copy(deg_sh.at[pl.ds(r, ZCH)],
                                out_hbm.at[c, pl.ds(r, ZCH)])

            return 0

        lax.fori_loop(0, KMAX, cout, 0)

    col4 = jnp.reshape(col, (NW, NIT // KB, KB, CH))
    return deg_kernel(col4, ones, zeros)


def _sc_scatter(y, row, col, zeros):
    mesh = plsc.VectorSubcoreMesh(core_axis_name="c", subcore_axis_name="s", num_cores=NC, num_subcores=NS)

    @functools.partial(
        pl.kernel,
        out_type=jax.ShapeDtypeStruct((NC, N, D), jnp.float32),
        mesh=mesh,
        scratch_types=[
            pltpu.VMEM((KB, CH), jnp.int32),
            pltpu.VMEM((KB, CH), jnp.int32),
            pltpu.VMEM((2, CH, D), jnp.float32),
            pltpu.VMEM((ZCS, D), jnp.float32),
            pltpu.VMEM_SHARED((N, D), jnp.float32),
            pltpu.SemaphoreType.DMA,
            pltpu.SemaphoreType.DMA,
        ],
    )
    def scat_kernel(y_hbm, row_hbm, col_hbm, zeros_hbm, out_hbm,
                    ridx_v, cidx_v, rows_v, zbuf_v, acc_sh, gsem, ssem):
        c = lax.axis_index("c")
        s = lax.axis_index("s")
        wid = s * NC + c

        pltpu.sync_copy(zeros_hbm, zbuf_v)

        def zchunk(k, _):
            ci = s + NS * k

            @pl.when(ci < N // ZCS)
            def _():
                pltpu.sync_copy(zbuf_v, acc_sh.at[pl.ds(ZCS * ci, ZCS)])

            return 0

        lax.fori_loop(0, -(-(N // ZCS) // NS), zchunk, 0)
        plsc.subcore_barrier()

        def edge(g, _):
            pltpu.sync_copy(row_hbm.at[wid, g], ridx_v)
            pltpu.sync_copy(col_hbm.at[wid, g], cidx_v)
            gcps = [pltpu.make_async_copy(y_hbm.at[ridx_v.at[j]],
                                          rows_v.at[j % 2], gsem)
                    for j in range(KB)]
            scps = [pltpu.make_async_copy(rows_v.at[j % 2],
                                          acc_sh.at[cidx_v.at[j]], ssem)
                    for j in range(KB)]
            gcps[0].start()
            gcps[1].start()
            for j in range(KB):
                gcps[j].wait()
                scps[j].start(add=True)
                scps[j].wait()
                if j + 2 < KB:
                    gcps[j + 2].start()
            return 0

        lax.fori_loop(0, NIT // KB, edge, 0)
        plsc.subcore_barrier()

        def cout(k, _):
            ci = s + NS * k

            @pl.when(ci < NCHK)
            def _():
                r = ZCH * ci
                pltpu.sync_copy(acc_sh.at[pl.ds(r, ZCH)],
                                out_hbm.at[c, pl.ds(r, ZCH)])

            return 0

        lax.fori_loop(0, KMAX, cout, 0)

    row4 = jnp.reshape(row, (NW, NIT // KB, KB, CH))
    col4 = jnp.reshape(col, (NW, NIT // KB, KB, CH))
    return scat_kernel(y, row4, col4, zeros[:ZCS])


_GRID = 10
_B = N // _GRID


def _dis_block(deg_ref):
    d = jnp.sum(deg_ref[0] + deg_ref[1], axis=1, keepdims=True) + 1.0
    return lax.rsqrt(d)


def _tc_xw(x, W):
    def body(x_ref, w_ref, y_ref):
        y_ref[...] = jnp.dot(x_ref[...], w_ref[...],
                             preferred_element_type=jnp.float32)

    return pl.pallas_call(
        body,
        grid=(_GRID,),
        in_specs=[
            pl.BlockSpec((_B, D), lambda i: (i, 0)),
            pl.BlockSpec((D, D), lambda i: (0, 0)),
        ],
        out_specs=pl.BlockSpec((_B, D), lambda i: (i, 0)),
        out_shape=jax.ShapeDtypeStruct((N, D), jnp.float32),
    )(x, W)


def _tc_scale(xw, deg2):
    def body(xw_ref, deg_ref, y_ref, dis_ref):
        dis = _dis_block(deg_ref)
        y_ref[...] = xw_ref[...] * dis
        dis_ref[...] = dis

    return pl.pallas_call(
        body,
        grid=(_GRID,),
        in_specs=[
            pl.BlockSpec((_B, D), lambda i: (i, 0)),
            pl.BlockSpec((NC, _B, DW), lambda i: (0, i, 0)),
        ],
        out_specs=[
            pl.BlockSpec((_B, D), lambda i: (i, 0)),
            pl.BlockSpec((_B, 1), lambda i: (i, 0)),
        ],
        out_shape=[
            jax.ShapeDtypeStruct((N, D), jnp.float32),
            jax.ShapeDtypeStruct((N, 1), jnp.float32),
        ],
    )(xw, deg2)


def _tc_finish(acc, y, dis, b2):
    def body(acc_ref, y_ref, dis_ref, b_ref, o_ref):
        su = ((acc_ref[0] + acc_ref[1] + y_ref[...]) * dis_ref[...]
              + b_ref[...])
        o_ref[...] = jnp.maximum(su, 0.0)

    return pl.pallas_call(
        body,
        grid=(_GRID,),
        in_specs=[
            pl.BlockSpec((NC, _B, D), lambda i: (0, i, 0)),
            pl.BlockSpec((_B, D), lambda i: (i, 0)),
            pl.BlockSpec((_B, 1), lambda i: (i, 0)),
            pl.BlockSpec((1, D), lambda i: (0, 0)),
        ],
        out_specs=pl.BlockSpec((_B, D), lambda i: (i, 0)),
        out_shape=jax.ShapeDtypeStruct((N, D), jnp.float32),
    )(acc, y, dis, b2)


def kernel(x, edge_index, W, b):
    row = edge_index[0].astype(jnp.int32)
    col = edge_index[1].astype(jnp.int32)
    ones = jnp.full((CH, DW), 1.0 / DW, jnp.float32)
    zeros = jnp.zeros((ZCH, D), jnp.float32)
    deg2 = _sc_degree(col, ones, zeros)
    xw = _tc_xw(x, W)
    y, dis = _tc_scale(xw, deg2)
    acc = _sc_scatter(y, row, col, zeros)
    return _tc_finish(acc, y, dis, jnp.reshape(b, (1, D)))


# speedup vs baseline: 27.8345x; 1.0582x over previous
import functools

import jax
import jax.numpy as jnp
from jax import lax
from jax.experimental import pallas as pl
from jax.experimental.pallas import tpu as pltpu
from jax.experimental.pallas import tpu_sc as plsc

N = 10000
E = 320000
D = 128
NC = 2
NS = 16
NW = NC * NS
EPW = E // NW
CH = 80
NIT = EPW // CH
KB = 5
ZCS = 80
ZCH = 200
NCHK = N // ZCH
KMAX = -(-NCHK // NS)
DW = 128
DWO = 16


def _sc_degree(col, ones, zeros):
    mesh = plsc.VectorSubcoreMesh(core_axis_name="c", subcore_axis_name="s", num_cores=NC, num_subcores=NS)

    @functools.partial(
        pl.kernel,
        out_type=jax.ShapeDtypeStruct((NC, N, DW), jnp.float32),
        mesh=mesh,
        scratch_types=[
            pltpu.VMEM((KB, CH), jnp.int32),
            pltpu.VMEM((CH, DW), jnp.float32),
            pltpu.VMEM((ZCH, DW), jnp.float32),
            pltpu.VMEM_SHARED((N, DW), jnp.float32),
            pltpu.SemaphoreType.DMA,
        ],
    )
    def deg_kernel(col_hbm, ones_hbm, zeros_hbm, out_hbm,
                   idx_v, ones_v, zbuf_v, deg_sh, fsem):
        c = lax.axis_index("c")
        s = lax.axis_index("s")
        wid = s * NC + c

        pltpu.sync_copy(ones_hbm, ones_v)
        pltpu.sync_copy(zeros_hbm, zbuf_v)

        def zchunk(k, _):
            ci = s + NS * k

            @pl.when(ci < NCHK)
            def _():
                pltpu.sync_copy(zbuf_v, deg_sh.at[pl.ds(ZCH * ci, ZCH)])

            return 0

        lax.fori_loop(0, KMAX, zchunk, 0)
        plsc.subcore_barrier()

        def edge(g, _):
            pltpu.sync_copy(col_hbm.at[wid, g], idx_v)
            cps = [pltpu.make_async_copy(ones_v, deg_sh.at[idx_v.at[j]], fsem)
                   for j in range(KB)]
            for cp in cps:
                cp.start(add=True)
            for cp in cps:
                cp.wait()
            return 0

        lax.fori_loop(0, NIT // KB, edge, 0)
        plsc.subcore_barrier()

        def cout(k, _):
            ci = s + NS * k

            @pl.when(ci < NCHK)
            def _():
                r = ZCH * ci
                pltpu.sync_copy(deg_sh.at[pl.ds(r, ZCH)],
                                out_hbm.at[c, pl.ds(r, ZCH)])

            return 0

        lax.fori_loop(0, KMAX, cout, 0)

    col4 = jnp.reshape(col, (NW, NIT // KB, KB, CH))
    return deg_kernel(col4, ones, zeros)


def _sc_scatter(y, row, col, zeros):
    mesh = plsc.VectorSubcoreMesh(core_axis_name="c", subcore_axis_name="s", num_cores=NC, num_subcores=NS)

    @functools.partial(
        pl.kernel,
        out_type=jax.ShapeDtypeStruct((NC, N, D), jnp.float32),
        mesh=mesh,
        scratch_types=[
            pltpu.VMEM((KB, CH), jnp.int32),
            pltpu.VMEM((KB, CH), jnp.int32),
            pltpu.VMEM((2, CH, D), jnp.float32),
            pltpu.VMEM((ZCS, D), jnp.float32),
            pltpu.VMEM_SHARED((N, D), jnp.float32),
            pltpu.SemaphoreType.DMA,
            pltpu.SemaphoreType.DMA,
        ],
    )
    def scat_kernel(y_hbm, row_hbm, col_hbm, zeros_hbm, out_hbm,
                    ridx_v, cidx_v, rows_v, zbuf_v, acc_sh, gsem, ssem):
        c = lax.axis_index("c")
        s = lax.axis_index("s")
        wid = s * NC + c

        pltpu.sync_copy(zeros_hbm, zbuf_v)

        def zchunk(k, _):
            ci = s + NS * k

            @pl.when(ci < N // ZCS)
            def _():
                pltpu.sync_copy(zbuf_v, acc_sh.at[pl.ds(ZCS * ci, ZCS)])

            return 0

        lax.fori_loop(0, -(-(N // ZCS) // NS), zchunk, 0)
        plsc.subcore_barrier()

        def edge(g, _):
            pltpu.sync_copy(row_hbm.at[wid, g], ridx_v)
            pltpu.sync_copy(col_hbm.at[wid, g], cidx_v)
            gcps = [pltpu.make_async_copy(y_hbm.at[ridx_v.at[j]],
                                          rows_v.at[j % 2], gsem)
                    for j in range(KB)]
            scps = [pltpu.make_async_copy(rows_v.at[j % 2],
                                          acc_sh.at[cidx_v.at[j]], ssem)
                    for j in range(KB)]
            gcps[0].start()
            gcps[1].start()
            for j in range(KB):
                gcps[j].wait()
                scps[j].start(add=True)
                if j + 2 < KB:
                    gcps[j + 2].start()
            for cp in scps:
                cp.wait()
            return 0

        lax.fori_loop(0, NIT // KB, edge, 0)
        plsc.subcore_barrier()

        def cout(k, _):
            ci = s + NS * k

            @pl.when(ci < NCHK)
            def _():
                r = ZCH * ci
                pltpu.sync_copy(acc_sh.at[pl.ds(r, ZCH)],
                                out_hbm.at[c, pl.ds(r, ZCH)])

            return 0

        lax.fori_loop(0, KMAX, cout, 0)

    row4 = jnp.reshape(row, (NW, NIT // KB, KB, CH))
    col4 = jnp.reshape(col, (NW, NIT // KB, KB, CH))
    return scat_kernel(y, row4, col4, zeros[:ZCS])


_GRID = 10
_B = N // _GRID


def _dis_block(deg_ref):
    d = jnp.sum(deg_ref[0] + deg_ref[1], axis=1, keepdims=True) + 1.0
    return lax.rsqrt(d)


def _tc_xw(x, W):
    def body(x_ref, w_ref, y_ref):
        y_ref[...] = jnp.dot(x_ref[...], w_ref[...],
                             preferred_element_type=jnp.float32)

    return pl.pallas_call(
        body,
        grid=(_GRID,),
        in_specs=[
            pl.BlockSpec((_B, D), lambda i: (i, 0)),
            pl.BlockSpec((D, D), lambda i: (0, 0)),
        ],
        out_specs=pl.BlockSpec((_B, D), lambda i: (i, 0)),
        out_shape=jax.ShapeDtypeStruct((N, D), jnp.float32),
    )(x, W)


def _tc_scale(xw, deg2):
    def body(xw_ref, deg_ref, y_ref, dis_ref):
        dis = _dis_block(deg_ref)
        y_ref[...] = xw_ref[...] * dis
        dis_ref[...] = dis

    return pl.pallas_call(
        body,
        grid=(_GRID,),
        in_specs=[
            pl.BlockSpec((_B, D), lambda i: (i, 0)),
            pl.BlockSpec((NC, _B, DW), lambda i: (0, i, 0)),
        ],
        out_specs=[
            pl.BlockSpec((_B, D), lambda i: (i, 0)),
            pl.BlockSpec((_B, 1), lambda i: (i, 0)),
        ],
        out_shape=[
            jax.ShapeDtypeStruct((N, D), jnp.float32),
            jax.ShapeDtypeStruct((N, 1), jnp.float32),
        ],
    )(xw, deg2)


def _tc_finish(acc, y, dis, b2):
    def body(acc_ref, y_ref, dis_ref, b_ref, o_ref):
        su = ((acc_ref[0] + acc_ref[1] + y_ref[...]) * dis_ref[...]
              + b_ref[...])
        o_ref[...] = jnp.maximum(su, 0.0)

    return pl.pallas_call(
        body,
        grid=(_GRID,),
        in_specs=[
            pl.BlockSpec((NC, _B, D), lambda i: (0, i, 0)),
            pl.BlockSpec((_B, D), lambda i: (i, 0)),
            pl.BlockSpec((_B, 1), lambda i: (i, 0)),
            pl.BlockSpec((1, D), lambda i: (0, 0)),
        ],
        out_specs=pl.BlockSpec((_B, D), lambda i: (i, 0)),
        out_shape=jax.ShapeDtypeStruct((N, D), jnp.float32),
    )(acc, y, dis, b2)


def kernel(x, edge_index, W, b):
    row = edge_index[0].astype(jnp.int32)
    col = edge_index[1].astype(jnp.int32)
    ones = jnp.full((CH, DW), 1.0 / DW, jnp.float32)
    zeros = jnp.zeros((ZCH, D), jnp.float32)
    deg2 = _sc_degree(col, ones, zeros)
    xw = _tc_xw(x, W)
    y, dis = _tc_scale(xw, deg2)
    acc = _sc_scatter(y, row, col, zeros)
    return _tc_finish(acc, y, dis, jnp.reshape(b, (1, D)))
